# Initial kernel scaffold; baseline (speedup 1.0000x reference)
import jax
import jax.numpy as jnp
from jax.experimental import pallas as pl


def kernel(x, edge_index, batch, x_b, edge_index_b, batch_b, lam, W1, b1, W2, b2, W3, b3, linW, linb):
    raise NotImplementedError("write your pallas kernel here")


# speedup vs baseline: 5.5546x; 5.5546x over previous
import functools

import jax
import jax.numpy as jnp
from jax import lax
from jax.experimental import pallas as pl
from jax.experimental.pallas import tpu as pltpu
from jax.experimental.pallas import tpu_sc as plsc

N = 10000
E = 320000
D = 128
H = 128
C = 10
G = 128

NC = 2
NS = 16
CH = 128
RPT = 640
N_PAD = NS * RPT
CPT = 160
EPT = CPT * CH
E_PAD = NS * EPT
ER = E_PAD // CH
QW = 32
NQ = D // QW

_mesh = plsc.VectorSubcoreMesh(core_axis_name="c", subcore_axis_name="s",
                               num_cores=NC, num_subcores=NS)


def _deg_body(dst_all, zlin, deg16, didx, hist):
    cid = lax.axis_index("c")
    sid = lax.axis_index("s")
    pltpu.sync_copy(dst_all.at[pl.ds(cid * ER + sid * CPT, CPT)], didx)
    pltpu.sync_copy(zlin, hist)

    ones16 = jnp.ones((16,), jnp.float32)

    def step(t, carry):
        j = t // 8
        v = (t % 8) * 16
        plsc.addupdate_scatter(hist, [didx[j, pl.ds(v, 16)]], ones16)
        return carry

    lax.fori_loop(0, CPT * 8, step, 0)
    pltpu.sync_copy(hist,
                    deg16.at[pl.ds((cid * NS + sid) * N_PAD, N_PAD)])


_sc_deg = functools.partial(
    pl.kernel,
    _deg_body,
    out_type=jax.ShapeDtypeStruct((NC * NS * N_PAD,), jnp.float32),
    mesh=_mesh,
    compiler_params=pltpu.CompilerParams(needs_layout_passes=False),
    scratch_types=[
        pltpu.VMEM((CPT, CH), jnp.int32),
        pltpu.VMEM((N_PAD,), jnp.float32),
    ],
)()


def _edge_body(u0, u1, u2, u3, src_all, dst_all, z32,
               a0, a1, a2, a3,
               sidx, didx, rows, zbuf, acc_sh, gsem):
    cid = lax.axis_index("c")
    sid = lax.axis_index("s")

    pltpu.sync_copy(src_all.at[pl.ds(cid * ER + sid * CPT, CPT)], sidx)
    pltpu.sync_copy(dst_all.at[pl.ds(cid * ER + sid * CPT, CPT)], didx)
    pltpu.sync_copy(z32, zbuf)

    for u_q, a_q in ((u0, a0), (u1, a1), (u2, a2), (u3, a3)):
        for i in range(RPT // CH):
            pltpu.sync_copy(zbuf, acc_sh.at[pl.ds(sid * RPT + i * CH, CH)])
        plsc.subcore_barrier()

        def chunk(j, c2, u_q=u_q):
            pltpu.async_copy(u_q.at[sidx.at[j]], rows, gsem).wait()
            pltpu.sync_copy(rows, acc_sh.at[didx.at[j]], add=True)
            return c2

        lax.fori_loop(0, CPT, chunk, 0)
        plsc.subcore_barrier()

        def outc(i, c2, a_q=a_q):
            r0 = sid * RPT + i * CH
            pltpu.sync_copy(acc_sh.at[pl.ds(r0, CH)], rows)
            pltpu.sync_copy(rows, a_q.at[pl.ds(cid * N_PAD + r0, CH)])
            return c2

        lax.fori_loop(0, RPT // CH, outc, 0)
        plsc.subcore_barrier()


_QSDS = jax.ShapeDtypeStruct((NC * N_PAD, QW), jnp.float32)
_sc_edge = functools.partial(
    pl.kernel,
    _edge_body,
    out_type=(_QSDS, _QSDS, _QSDS, _QSDS),
    mesh=_mesh,
    compiler_params=pltpu.CompilerParams(use_tc_tiling_on_sc=False),
    scratch_types=[
        pltpu.VMEM((CPT, CH), jnp.int32),
        pltpu.VMEM((CPT, CH), jnp.int32),
        pltpu.VMEM((CH, QW), jnp.float32),
        pltpu.VMEM((CH, QW), jnp.float32),
        pltpu.VMEM_SHARED((N_PAD, QW), jnp.float32),
        pltpu.SemaphoreType.DMA,
    ],
)()


_TBR = 1280
_TGRID = NC * N_PAD // _TBR


_QSPEC = pl.BlockSpec((_TBR, QW), lambda i: (i, 0))
_QOUT4 = [_QSPEC] * NQ
_QSDS4 = [_QSDS] * NQ


def _write_quarters(refs, val):
    for q, r in enumerate(refs):
        r[...] = val[:, q * QW:(q + 1) * QW]


def _tc_prep_body(x_ref, deg16_ref, w_ref, o0, o1, o2, o3, deg_ref):
    deg = jnp.sum(deg16_ref[...], axis=0)[:, None]
    deg_ref[...] = deg
    dis = lax.rsqrt(deg + 1.0)
    u = jnp.dot(x_ref[...], w_ref[...],
                preferred_element_type=jnp.float32) * dis
    _write_quarters((o0, o1, o2, o3), u)


def _tc_prep(xp, deg16, w):
    return pl.pallas_call(
        _tc_prep_body,
        grid=(_TGRID,),
        in_specs=[
            pl.BlockSpec((_TBR, D), lambda i: (i, 0)),
            pl.BlockSpec((NS, _TBR), lambda i: (i // 8, i % 8)),
            pl.BlockSpec((D, H), lambda i: (0, 0)),
        ],
        out_specs=_QOUT4 + [pl.BlockSpec((_TBR, 1), lambda i: (i, 0))],
        out_shape=_QSDS4 + [
            jax.ShapeDtypeStruct((NC * N_PAD, 1), jnp.float32)],
    )(xp, deg16, w)


def _relu_gcn(accs, us, deg, b):
    dis = lax.rsqrt(deg + 1.0)
    full = jnp.concatenate(
        [a[...] + u[...] for a, u in zip(accs, us)], axis=1)
    return jnp.maximum(full * dis + b, 0.0)


def _tc_mid_body(a0, a1, a2, a3, v0, v1, v2, v3, deg_ref, b_ref, w_ref,
                 o0, o1, o2, o3):
    dis = lax.rsqrt(deg_ref[...] + 1.0)
    h = _relu_gcn((a0, a1, a2, a3), (v0, v1, v2, v3), deg_ref[...], b_ref[...])
    u = jnp.dot(h, w_ref[...], preferred_element_type=jnp.float32) * dis
    _write_quarters((o0, o1, o2, o3), u)


def _tc_mid(accs, us, deg, b, w):
    return pl.pallas_call(
        _tc_mid_body,
        grid=(_TGRID,),
        in_specs=[_QSPEC] * (2 * NQ) + [
            pl.BlockSpec((_TBR, 1), lambda i: (i, 0)),
            pl.BlockSpec((1, H), lambda i: (0, 0)),
            pl.BlockSpec((H, H), lambda i: (0, 0)),
        ],
        out_specs=_QOUT4,
        out_shape=_QSDS4,
    )(*accs, *us, deg, b, w)


def _tc_lastpool_body(a0, a1, a2, a3, v0, v1, v2, v3, deg_ref, b_ref, bat_ref,
                      pool_ref, cnt_ref):
    i = pl.program_id(0)
    h = _relu_gcn((a0, a1, a2, a3), (v0, v1, v2, v3), deg_ref[...], b_ref[...])
    cols = lax.broadcasted_iota(jnp.int32, (_TBR, NC * G), 1)
    onehot = (bat_ref[...] == cols).astype(jnp.float32)
    pool_part = lax.dot_general(onehot, h, (((0,), (0,)), ((), ())),
                                preferred_element_type=jnp.float32)
    cnt_part = lax.dot_general(onehot, jnp.ones((_TBR, 8), jnp.float32),
                               (((0,), (0,)), ((), ())),
                               preferred_element_type=jnp.float32)

    @pl.when(i == 0)
    def _():
        pool_ref[...] = jnp.zeros_like(pool_ref)
        cnt_ref[...] = jnp.zeros_like(cnt_ref)

    pool_ref[...] += pool_part
    cnt_ref[...] += cnt_part


def _tc_lastpool(accs, us, deg, b, bat):
    return pl.pallas_call(
        _tc_lastpool_body,
        grid=(_TGRID,),
        in_specs=[_QSPEC] * (2 * NQ) + [
            pl.BlockSpec((_TBR, 1), lambda i: (i, 0)),
            pl.BlockSpec((1, H), lambda i: (0, 0)),
            pl.BlockSpec((_TBR, 1), lambda i: (i, 0)),
        ],
        out_specs=[
            pl.BlockSpec((NC * G, H), lambda i: (0, 0)),
            pl.BlockSpec((NC * G, 8), lambda i: (0, 0)),
        ],
        out_shape=[
            jax.ShapeDtypeStruct((NC * G, H), jnp.float32),
            jax.ShapeDtypeStruct((NC * G, 8), jnp.float32),
        ],
    )(*accs, *us, deg, b, bat)


def _tc_final_body(pa_ref, ca_ref, pb_ref, cb_ref, lam_ref, w_ref, b_ref,
                   o_ref):
    lam = lam_ref[0, 0]
    ga = pa_ref[...] / jnp.maximum(ca_ref[:, 0:1], 1.0)
    gb = pb_ref[...] / jnp.maximum(cb_ref[:, 0:1], 1.0)
    m = lam * ga + (1.0 - lam) * gb
    o = jnp.dot(m, w_ref[...], preferred_element_type=jnp.float32) + b_ref[...]
    col = lax.broadcasted_iota(jnp.int32, o.shape, 1)
    om = jnp.where(col < C, o, -1e30)
    mx = jnp.max(om, axis=1, keepdims=True)
    lse = mx + jnp.log(jnp.sum(jnp.exp(om - mx), axis=1, keepdims=True))
    o_ref[...] = o - lse


def _tc_final(pa, ca, pb, cb, lam2, wpad, bpad):
    return pl.pallas_call(
        _tc_final_body,
        in_specs=[
            pl.BlockSpec((G, H), lambda: (0, 0)),
            pl.BlockSpec((G, 8), lambda: (0, 0)),
            pl.BlockSpec((G, H), lambda: (0, 0)),
            pl.BlockSpec((G, 8), lambda: (0, 0)),
            pl.BlockSpec(memory_space=pltpu.SMEM),
            pl.BlockSpec((H, H), lambda: (0, 0)),
            pl.BlockSpec((1, H), lambda: (0, 0)),
        ],
        out_specs=pl.BlockSpec((G, H), lambda: (0, 0)),
        out_shape=jax.ShapeDtypeStruct((G, H), jnp.float32),
    )(pa, ca, pb, cb, lam2, wpad, bpad)


def kernel(x, edge_index, batch, x_b, edge_index_b, batch_b, lam,
           W1, b1, W2, b2, W3, b3, linW, linb):
    f32 = jnp.float32
    i32 = jnp.int32

    zpadn = jnp.zeros((N_PAD - N, D), f32)
    x_all = jnp.concatenate([x, zpadn, x_b, zpadn])

    def pad_edges(ei, src_off):
        src = jnp.concatenate(
            [ei[0] + i32(src_off), jnp.full((E_PAD - E,), src_off, i32)])
        dst = jnp.concatenate([ei[1], jnp.full((E_PAD - E,), N, i32)])
        return src.reshape(ER, CH), dst.reshape(ER, CH)

    srca, dsta = pad_edges(edge_index, 0)
    srcb, dstb = pad_edges(edge_index_b, N_PAD)
    src_all = jnp.concatenate([srca, srcb])
    dst_all = jnp.concatenate([dsta, dstb])

    bpadv = jnp.full((N_PAD - N,), NC * G, i32)
    bat2d = jnp.concatenate(
        [batch, bpadv, batch_b + i32(G), bpadv]).reshape(NC * N_PAD, 1)

    zlin = jnp.zeros((N_PAD,), f32)
    z32 = jnp.zeros((CH, QW), f32)

    deg16 = _sc_deg(dst_all, zlin).reshape(NC * NS, N_PAD)

    *u1, deg2d = _tc_prep(x_all, deg16, W1)
    acc1 = _sc_edge(*u1, src_all, dst_all, z32)

    b1r, b2r, b3r = b1.reshape(1, H), b2.reshape(1, H), b3.reshape(1, H)
    u2 = _tc_mid(acc1, u1, deg2d, b1r, W2)
    acc2 = _sc_edge(*u2, src_all, dst_all, z32)

    u3 = _tc_mid(acc2, u2, deg2d, b2r, W3)
    acc3 = _sc_edge(*u3, src_all, dst_all, z32)

    pool_all, cnt_all = _tc_lastpool(acc3, u3, deg2d, b3r, bat2d)

    wpad = jnp.pad(linW, ((0, 0), (0, H - C)))
    bp = jnp.pad(linb, (0, H - C)).reshape(1, H)
    lam2 = jnp.reshape(lam, (1, 1)).astype(f32)

    out = _tc_final(pool_all[:G], cnt_all[:G],
                    pool_all[G:], cnt_all[G:], lam2, wpad, bp)
    return out[:, :C]


# speedup vs baseline: 7.8620x; 1.4154x over previous
import functools

import jax
import jax.numpy as jnp
from jax import lax
from jax.experimental import pallas as pl
from jax.experimental.pallas import tpu as pltpu
from jax.experimental.pallas import tpu_sc as plsc

N = 10000
E = 320000
D = 128
H = 128
C = 10
G = 128

NC = 2
NS = 16
CH = 128
RPT = 640
N_PAD = NS * RPT
CPT = 160
EPT = CPT * CH
E_PAD = NS * EPT
ER = E_PAD // CH
QW = 32
NQ = D // QW

_mesh = plsc.VectorSubcoreMesh(core_axis_name="c", subcore_axis_name="s",
                               num_cores=NC, num_subcores=NS)


def _deg_body(dst_all, zlin, deg16, didx, hist):
    cid = lax.axis_index("c")
    sid = lax.axis_index("s")
    pltpu.sync_copy(dst_all.at[pl.ds(cid * ER + sid * CPT, CPT)], didx)
    pltpu.sync_copy(zlin, hist)

    ones16 = jnp.ones((16,), jnp.float32)

    def step(t, carry):
        j = t // 8
        v = (t % 8) * 16
        plsc.addupdate_scatter(hist, [didx[j, pl.ds(v, 16)]], ones16)
        return carry

    lax.fori_loop(0, CPT * 8, step, 0)
    pltpu.sync_copy(hist,
                    deg16.at[pl.ds((cid * NS + sid) * N_PAD, N_PAD)])


_sc_deg = functools.partial(
    pl.kernel,
    _deg_body,
    out_type=jax.ShapeDtypeStruct((NC * NS * N_PAD,), jnp.float32),
    mesh=_mesh,
    compiler_params=pltpu.CompilerParams(needs_layout_passes=False),
    scratch_types=[
        pltpu.VMEM((CPT, CH), jnp.int32),
        pltpu.VMEM((N_PAD,), jnp.float32),
    ],
)()


GRP = 4


def _edge_body(u0, u1, u2, u3, src_all, dst_all, z32,
               a0, a1, a2, a3,
               sidx, didx, rows_a, rows_b, zbuf, acc_sh,
               gsem_a, gsem_b, ssem):
    cid = lax.axis_index("c")
    sid = lax.axis_index("s")

    pltpu.sync_copy(src_all.at[pl.ds(cid * ER + sid * CPT, CPT)], sidx)
    pltpu.sync_copy(dst_all.at[pl.ds(cid * ER + sid * CPT, CPT)], didx)
    pltpu.sync_copy(z32, zbuf)

    for u_q, a_q in ((u0, a0), (u1, a1), (u2, a2), (u3, a3)):
        for i in range(RPT // CH):
            pltpu.sync_copy(zbuf, acc_sh.at[pl.ds(sid * RPT + i * CH, CH)])
        plsc.subcore_barrier()

        def chunk(it, c2, u_q=u_q):
            j0 = it * (2 * GRP)
            ga = [pltpu.async_copy(u_q.at[sidx.at[j0 + k]],
                                   rows_a.at[k], gsem_a)
                  for k in range(GRP)]
            gb = [pltpu.async_copy(u_q.at[sidx.at[j0 + GRP + k]],
                                   rows_b.at[k], gsem_b)
                  for k in range(GRP)]
            sds = []
            for d in ga:
                d.wait()
            for k in range(GRP):
                sds.append(pltpu.async_copy(
                    rows_a.at[k], acc_sh.at[didx.at[j0 + k]], ssem, add=True))
            for d in gb:
                d.wait()
            for k in range(GRP):
                sds.append(pltpu.async_copy(
                    rows_b.at[k], acc_sh.at[didx.at[j0 + GRP + k]], ssem,
                    add=True))
            for d in sds:
                d.wait()
            return c2

        lax.fori_loop(0, CPT // (2 * GRP), chunk, 0)
        plsc.subcore_barrier()

        def outc(i, c2, a_q=a_q):
            r0 = sid * RPT + i * CH
            pltpu.sync_copy(acc_sh.at[pl.ds(r0, CH)], rows_a.at[0])
            pltpu.sync_copy(rows_a.at[0], a_q.at[pl.ds(cid * N_PAD + r0, CH)])
            return c2

        lax.fori_loop(0, RPT // CH, outc, 0)
        plsc.subcore_barrier()


_QSDS = jax.ShapeDtypeStruct((NC * N_PAD, QW), jnp.float32)
_sc_edge = functools.partial(
    pl.kernel,
    _edge_body,
    out_type=(_QSDS, _QSDS, _QSDS, _QSDS),
    mesh=_mesh,
    compiler_params=pltpu.CompilerParams(use_tc_tiling_on_sc=False),
    scratch_types=[
        pltpu.VMEM((CPT, CH), jnp.int32),
        pltpu.VMEM((CPT, CH), jnp.int32),
        pltpu.VMEM((GRP, CH, QW), jnp.float32),
        pltpu.VMEM((GRP, CH, QW), jnp.float32),
        pltpu.VMEM((CH, QW), jnp.float32),
        pltpu.VMEM_SHARED((N_PAD, QW), jnp.float32),
        pltpu.SemaphoreType.DMA,
        pltpu.SemaphoreType.DMA,
        pltpu.SemaphoreType.DMA,
    ],
)()


_TBR = 1280
_TGRID = NC * N_PAD // _TBR


_QSPEC = pl.BlockSpec((_TBR, QW), lambda i: (i, 0))
_QOUT4 = [_QSPEC] * NQ
_QSDS4 = [_QSDS] * NQ


def _write_quarters(refs, val):
    for q, r in enumerate(refs):
        r[...] = val[:, q * QW:(q + 1) * QW]


def _tc_prep_body(x_ref, deg16_ref, w_ref, o0, o1, o2, o3, deg_ref):
    deg = jnp.sum(deg16_ref[...], axis=0)[:, None]
    deg_ref[...] = deg
    dis = lax.rsqrt(deg + 1.0)
    u = jnp.dot(x_ref[...], w_ref[...],
                preferred_element_type=jnp.float32) * dis
    _write_quarters((o0, o1, o2, o3), u)


def _tc_prep(xp, deg16, w):
    return pl.pallas_call(
        _tc_prep_body,
        grid=(_TGRID,),
        in_specs=[
            pl.BlockSpec((_TBR, D), lambda i: (i, 0)),
            pl.BlockSpec((NS, _TBR), lambda i: (i // 8, i % 8)),
            pl.BlockSpec((D, H), lambda i: (0, 0)),
        ],
        out_specs=_QOUT4 + [pl.BlockSpec((_TBR, 1), lambda i: (i, 0))],
        out_shape=_QSDS4 + [
            jax.ShapeDtypeStruct((NC * N_PAD, 1), jnp.float32)],
    )(xp, deg16, w)


def _relu_gcn(accs, us, deg, b):
    dis = lax.rsqrt(deg + 1.0)
    full = jnp.concatenate(
        [a[...] + u[...] for a, u in zip(accs, us)], axis=1)
    return jnp.maximum(full * dis + b, 0.0)


def _tc_mid_body(a0, a1, a2, a3, v0, v1, v2, v3, deg_ref, b_ref, w_ref,
                 o0, o1, o2, o3):
    dis = lax.rsqrt(deg_ref[...] + 1.0)
    h = _relu_gcn((a0, a1, a2, a3), (v0, v1, v2, v3), deg_ref[...], b_ref[...])
    u = jnp.dot(h, w_ref[...], preferred_element_type=jnp.float32) * dis
    _write_quarters((o0, o1, o2, o3), u)


def _tc_mid(accs, us, deg, b, w):
    return pl.pallas_call(
        _tc_mid_body,
        grid=(_TGRID,),
        in_specs=[_QSPEC] * (2 * NQ) + [
            pl.BlockSpec((_TBR, 1), lambda i: (i, 0)),
            pl.BlockSpec((1, H), lambda i: (0, 0)),
            pl.BlockSpec((H, H), lambda i: (0, 0)),
        ],
        out_specs=_QOUT4,
        out_shape=_QSDS4,
    )(*accs, *us, deg, b, w)


def _tc_lastpool_body(a0, a1, a2, a3, v0, v1, v2, v3, deg_ref, b_ref, bat_ref,
                      pool_ref, cnt_ref):
    i = pl.program_id(0)
    h = _relu_gcn((a0, a1, a2, a3), (v0, v1, v2, v3), deg_ref[...], b_ref[...])
    cols = lax.broadcasted_iota(jnp.int32, (_TBR, NC * G), 1)
    onehot = (bat_ref[...] == cols).astype(jnp.float32)
    pool_part = lax.dot_general(onehot, h, (((0,), (0,)), ((), ())),
                                preferred_element_type=jnp.float32)
    cnt_part = lax.dot_general(onehot, jnp.ones((_TBR, 8), jnp.float32),
                               (((0,), (0,)), ((), ())),
                               preferred_element_type=jnp.float32)

    @pl.when(i == 0)
    def _():
        pool_ref[...] = jnp.zeros_like(pool_ref)
        cnt_ref[...] = jnp.zeros_like(cnt_ref)

    pool_ref[...] += pool_part
    cnt_ref[...] += cnt_part


def _tc_lastpool(accs, us, deg, b, bat):
    return pl.pallas_call(
        _tc_lastpool_body,
        grid=(_TGRID,),
        in_specs=[_QSPEC] * (2 * NQ) + [
            pl.BlockSpec((_TBR, 1), lambda i: (i, 0)),
            pl.BlockSpec((1, H), lambda i: (0, 0)),
            pl.BlockSpec((_TBR, 1), lambda i: (i, 0)),
        ],
        out_specs=[
            pl.BlockSpec((NC * G, H), lambda i: (0, 0)),
            pl.BlockSpec((NC * G, 8), lambda i: (0, 0)),
        ],
        out_shape=[
            jax.ShapeDtypeStruct((NC * G, H), jnp.float32),
            jax.ShapeDtypeStruct((NC * G, 8), jnp.float32),
        ],
    )(*accs, *us, deg, b, bat)


def _tc_final_body(pa_ref, ca_ref, pb_ref, cb_ref, lam_ref, w_ref, b_ref,
                   o_ref):
    lam = lam_ref[0, 0]
    ga = pa_ref[...] / jnp.maximum(ca_ref[:, 0:1], 1.0)
    gb = pb_ref[...] / jnp.maximum(cb_ref[:, 0:1], 1.0)
    m = lam * ga + (1.0 - lam) * gb
    o = jnp.dot(m, w_ref[...], preferred_element_type=jnp.float32) + b_ref[...]
    col = lax.broadcasted_iota(jnp.int32, o.shape, 1)
    om = jnp.where(col < C, o, -1e30)
    mx = jnp.max(om, axis=1, keepdims=True)
    lse = mx + jnp.log(jnp.sum(jnp.exp(om - mx), axis=1, keepdims=True))
    o_ref[...] = o - lse


def _tc_final(pa, ca, pb, cb, lam2, wpad, bpad):
    return pl.pallas_call(
        _tc_final_body,
        in_specs=[
            pl.BlockSpec((G, H), lambda: (0, 0)),
            pl.BlockSpec((G, 8), lambda: (0, 0)),
            pl.BlockSpec((G, H), lambda: (0, 0)),
            pl.BlockSpec((G, 8), lambda: (0, 0)),
            pl.BlockSpec(memory_space=pltpu.SMEM),
            pl.BlockSpec((H, H), lambda: (0, 0)),
            pl.BlockSpec((1, H), lambda: (0, 0)),
        ],
        out_specs=pl.BlockSpec((G, H), lambda: (0, 0)),
        out_shape=jax.ShapeDtypeStruct((G, H), jnp.float32),
    )(pa, ca, pb, cb, lam2, wpad, bpad)


def kernel(x, edge_index, batch, x_b, edge_index_b, batch_b, lam,
           W1, b1, W2, b2, W3, b3, linW, linb):
    f32 = jnp.float32
    i32 = jnp.int32

    zpadn = jnp.zeros((N_PAD - N, D), f32)
    x_all = jnp.concatenate([x, zpadn, x_b, zpadn])

    def pad_edges(ei, src_off):
        src = jnp.concatenate(
            [ei[0] + i32(src_off), jnp.full((E_PAD - E,), src_off, i32)])
        dst = jnp.concatenate([ei[1], jnp.full((E_PAD - E,), N, i32)])
        return src.reshape(ER, CH), dst.reshape(ER, CH)

    srca, dsta = pad_edges(edge_index, 0)
    srcb, dstb = pad_edges(edge_index_b, N_PAD)
    src_all = jnp.concatenate([srca, srcb])
    dst_all = jnp.concatenate([dsta, dstb])

    bpadv = jnp.full((N_PAD - N,), NC * G, i32)
    bat2d = jnp.concatenate(
        [batch, bpadv, batch_b + i32(G), bpadv]).reshape(NC * N_PAD, 1)

    zlin = jnp.zeros((N_PAD,), f32)
    z32 = jnp.zeros((CH, QW), f32)

    deg16 = _sc_deg(dst_all, zlin).reshape(NC * NS, N_PAD)

    *u1, deg2d = _tc_prep(x_all, deg16, W1)
    acc1 = _sc_edge(*u1, src_all, dst_all, z32)

    b1r, b2r, b3r = b1.reshape(1, H), b2.reshape(1, H), b3.reshape(1, H)
    u2 = _tc_mid(acc1, u1, deg2d, b1r, W2)
    acc2 = _sc_edge(*u2, src_all, dst_all, z32)

    u3 = _tc_mid(acc2, u2, deg2d, b2r, W3)
    acc3 = _sc_edge(*u3, src_all, dst_all, z32)

    pool_all, cnt_all = _tc_lastpool(acc3, u3, deg2d, b3r, bat2d)

    wpad = jnp.pad(linW, ((0, 0), (0, H - C)))
    bp = jnp.pad(linb, (0, H - C)).reshape(1, H)
    lam2 = jnp.reshape(lam, (1, 1)).astype(f32)

    out = _tc_final(pool_all[:G], cnt_all[:G],
                    pool_all[G:], cnt_all[G:], lam2, wpad, bp)
    return out[:, :C]


# speedup vs baseline: 14.9650x; 1.9035x over previous
import functools

import jax
import jax.numpy as jnp
from jax import lax
from jax.experimental import pallas as pl
from jax.experimental.pallas import tpu as pltpu
from jax.experimental.pallas import tpu_sc as plsc

N = 10000
E = 320000
D = 128
H = 128
C = 10
G = 128

NC = 2
NS = 16
CH = 128
RPT = 640
N_PAD = NS * RPT
CPT = 160
EPT = CPT * CH
E_PAD = NS * EPT
ER = E_PAD // CH
QW = 64
NQ = D // QW

_mesh = plsc.VectorSubcoreMesh(core_axis_name="c", subcore_axis_name="s",
                               num_cores=NC, num_subcores=NS)


def _deg_body(dst_all, zlin, deg16, didx, hist):
    cid = lax.axis_index("c")
    sid = lax.axis_index("s")
    pltpu.sync_copy(dst_all.at[pl.ds(cid * ER + sid * CPT, CPT)], didx)
    pltpu.sync_copy(zlin, hist)

    ones16 = jnp.ones((16,), jnp.float32)

    def step(t, carry):
        j = t // 8
        v = (t % 8) * 16
        plsc.addupdate_scatter(hist, [didx[j, pl.ds(v, 16)]], ones16)
        return carry

    lax.fori_loop(0, CPT * 8, step, 0)
    pltpu.sync_copy(hist,
                    deg16.at[pl.ds((cid * NS + sid) * N_PAD, N_PAD)])


_sc_deg = functools.partial(
    pl.kernel,
    _deg_body,
    out_type=jax.ShapeDtypeStruct((NC * NS * N_PAD,), jnp.float32),
    mesh=_mesh,
    compiler_params=pltpu.CompilerParams(needs_layout_passes=False),
    scratch_types=[
        pltpu.VMEM((CPT, CH), jnp.int32),
        pltpu.VMEM((N_PAD,), jnp.float32),
    ],
)()


GRP = 4


def _edge_body(u0, u1, src_all, dst_all, z32,
               a0, a1,
               sidx, didx, rows_a, rows_b, zbuf, acc_sh,
               gsem_a, gsem_b, ssem):
    cid = lax.axis_index("c")
    sid = lax.axis_index("s")

    pltpu.sync_copy(src_all.at[pl.ds(cid * ER + sid * CPT, CPT)], sidx)
    pltpu.sync_copy(dst_all.at[pl.ds(cid * ER + sid * CPT, CPT)], didx)
    pltpu.sync_copy(z32, zbuf)

    for u_q, a_q in ((u0, a0), (u1, a1)):
        for i in range(RPT // CH):
            pltpu.sync_copy(zbuf, acc_sh.at[pl.ds(sid * RPT + i * CH, CH)])
        plsc.subcore_barrier()

        def chunk(it, c2, u_q=u_q):
            j0 = it * (2 * GRP)
            ga = [pltpu.async_copy(u_q.at[sidx.at[j0 + k]],
                                   rows_a.at[k], gsem_a)
                  for k in range(GRP)]
            gb = [pltpu.async_copy(u_q.at[sidx.at[j0 + GRP + k]],
                                   rows_b.at[k], gsem_b)
                  for k in range(GRP)]
            sds = []
            for d in ga:
                d.wait()
            for k in range(GRP):
                sds.append(pltpu.async_copy(
                    rows_a.at[k], acc_sh.at[didx.at[j0 + k]], ssem, add=True))
            for d in gb:
                d.wait()
            for k in range(GRP):
                sds.append(pltpu.async_copy(
                    rows_b.at[k], acc_sh.at[didx.at[j0 + GRP + k]], ssem,
                    add=True))
            for d in sds:
                d.wait()
            return c2

        lax.fori_loop(0, CPT // (2 * GRP), chunk, 0)
        plsc.subcore_barrier()

        def outc(i, c2, a_q=a_q):
            r0 = sid * RPT + i * CH
            pltpu.sync_copy(acc_sh.at[pl.ds(r0, CH)], rows_a.at[0])
            pltpu.sync_copy(rows_a.at[0], a_q.at[pl.ds(cid * N_PAD + r0, CH)])
            return c2

        lax.fori_loop(0, RPT // CH, outc, 0)
        plsc.subcore_barrier()


_QSDS = jax.ShapeDtypeStruct((NC * N_PAD, QW), jnp.bfloat16)
_sc_edge = functools.partial(
    pl.kernel,
    _edge_body,
    out_type=(_QSDS, _QSDS),
    mesh=_mesh,
    compiler_params=pltpu.CompilerParams(use_tc_tiling_on_sc=False),
    scratch_types=[
        pltpu.VMEM((CPT, CH), jnp.int32),
        pltpu.VMEM((CPT, CH), jnp.int32),
        pltpu.VMEM((GRP, CH, QW), jnp.bfloat16),
        pltpu.VMEM((GRP, CH, QW), jnp.bfloat16),
        pltpu.VMEM((CH, QW), jnp.bfloat16),
        pltpu.VMEM_SHARED((N_PAD, QW), jnp.bfloat16),
        pltpu.SemaphoreType.DMA,
        pltpu.SemaphoreType.DMA,
        pltpu.SemaphoreType.DMA,
    ],
)()


_TBR = 1280
_TGRID = NC * N_PAD // _TBR


_QSPEC = pl.BlockSpec((_TBR, QW), lambda i: (i, 0))
_QOUT4 = [_QSPEC] * NQ
_QSDS4 = [_QSDS] * NQ


def _write_quarters(refs, val):
    for q, r in enumerate(refs):
        r[...] = val[:, q * QW:(q + 1) * QW].astype(jnp.bfloat16)


def _tc_prep_body(x_ref, deg16_ref, w_ref, o0, o1, deg_ref):
    deg = jnp.sum(deg16_ref[...], axis=0)[:, None]
    deg_ref[...] = deg
    dis = lax.rsqrt(deg + 1.0)
    u = jnp.dot(x_ref[...], w_ref[...],
                preferred_element_type=jnp.float32) * dis
    _write_quarters((o0, o1), u)


def _tc_prep(xp, deg16, w):
    return pl.pallas_call(
        _tc_prep_body,
        grid=(_TGRID,),
        in_specs=[
            pl.BlockSpec((_TBR, D), lambda i: (i, 0)),
            pl.BlockSpec((NS, _TBR), lambda i: (i // 8, i % 8)),
            pl.BlockSpec((D, H), lambda i: (0, 0)),
        ],
        out_specs=_QOUT4 + [pl.BlockSpec((_TBR, 1), lambda i: (i, 0))],
        out_shape=_QSDS4 + [
            jax.ShapeDtypeStruct((NC * N_PAD, 1), jnp.float32)],
    )(xp, deg16, w)


def _relu_gcn(accs, us, deg, b):
    dis = lax.rsqrt(deg + 1.0)
    full = jnp.concatenate(
        [a[...].astype(jnp.float32) + u[...].astype(jnp.float32)
         for a, u in zip(accs, us)], axis=1)
    return jnp.maximum(full * dis + b, 0.0)


def _tc_mid_body(a0, a1, v0, v1, deg_ref, b_ref, w_ref,
                 o0, o1):
    dis = lax.rsqrt(deg_ref[...] + 1.0)
    h = _relu_gcn((a0, a1), (v0, v1), deg_ref[...], b_ref[...])
    u = jnp.dot(h, w_ref[...], preferred_element_type=jnp.float32) * dis
    _write_quarters((o0, o1), u)


def _tc_mid(accs, us, deg, b, w):
    return pl.pallas_call(
        _tc_mid_body,
        grid=(_TGRID,),
        in_specs=[_QSPEC] * (2 * NQ) + [
            pl.BlockSpec((_TBR, 1), lambda i: (i, 0)),
            pl.BlockSpec((1, H), lambda i: (0, 0)),
            pl.BlockSpec((H, H), lambda i: (0, 0)),
        ],
        out_specs=_QOUT4,
        out_shape=_QSDS4,
    )(*accs, *us, deg, b, w)


def _tc_lastpool_body(a0, a1, v0, v1, deg_ref, b_ref, bat_ref,
                      pool_ref, cnt_ref):
    i = pl.program_id(0)
    h = _relu_gcn((a0, a1), (v0, v1), deg_ref[...], b_ref[...])
    cols = lax.broadcasted_iota(jnp.int32, (_TBR, NC * G), 1)
    onehot = (bat_ref[...] == cols).astype(jnp.float32)
    pool_part = lax.dot_general(onehot, h, (((0,), (0,)), ((), ())),
                                preferred_element_type=jnp.float32)
    cnt_part = lax.dot_general(onehot, jnp.ones((_TBR, 8), jnp.float32),
                               (((0,), (0,)), ((), ())),
                               preferred_element_type=jnp.float32)

    @pl.when(i == 0)
    def _():
        pool_ref[...] = jnp.zeros_like(pool_ref)
        cnt_ref[...] = jnp.zeros_like(cnt_ref)

    pool_ref[...] += pool_part
    cnt_ref[...] += cnt_part


def _tc_lastpool(accs, us, deg, b, bat):
    return pl.pallas_call(
        _tc_lastpool_body,
        grid=(_TGRID,),
        in_specs=[_QSPEC] * (2 * NQ) + [
            pl.BlockSpec((_TBR, 1), lambda i: (i, 0)),
            pl.BlockSpec((1, H), lambda i: (0, 0)),
            pl.BlockSpec((_TBR, 1), lambda i: (i, 0)),
        ],
        out_specs=[
            pl.BlockSpec((NC * G, H), lambda i: (0, 0)),
            pl.BlockSpec((NC * G, 8), lambda i: (0, 0)),
        ],
        out_shape=[
            jax.ShapeDtypeStruct((NC * G, H), jnp.float32),
            jax.ShapeDtypeStruct((NC * G, 8), jnp.float32),
        ],
    )(*accs, *us, deg, b, bat)


def _tc_final_body(pa_ref, ca_ref, pb_ref, cb_ref, lam_ref, w_ref, b_ref,
                   o_ref):
    lam = lam_ref[0, 0]
    ga = pa_ref[...] / jnp.maximum(ca_ref[:, 0:1], 1.0)
    gb = pb_ref[...] / jnp.maximum(cb_ref[:, 0:1], 1.0)
    m = lam * ga + (1.0 - lam) * gb
    o = jnp.dot(m, w_ref[...], preferred_element_type=jnp.float32) + b_ref[...]
    col = lax.broadcasted_iota(jnp.int32, o.shape, 1)
    om = jnp.where(col < C, o, -1e30)
    mx = jnp.max(om, axis=1, keepdims=True)
    lse = mx + jnp.log(jnp.sum(jnp.exp(om - mx), axis=1, keepdims=True))
    o_ref[...] = o - lse


def _tc_final(pa, ca, pb, cb, lam2, wpad, bpad):
    return pl.pallas_call(
        _tc_final_body,
        in_specs=[
            pl.BlockSpec((G, H), lambda: (0, 0)),
            pl.BlockSpec((G, 8), lambda: (0, 0)),
            pl.BlockSpec((G, H), lambda: (0, 0)),
            pl.BlockSpec((G, 8), lambda: (0, 0)),
            pl.BlockSpec(memory_space=pltpu.SMEM),
            pl.BlockSpec((H, H), lambda: (0, 0)),
            pl.BlockSpec((1, H), lambda: (0, 0)),
        ],
        out_specs=pl.BlockSpec((G, H), lambda: (0, 0)),
        out_shape=jax.ShapeDtypeStruct((G, H), jnp.float32),
    )(pa, ca, pb, cb, lam2, wpad, bpad)


def kernel(x, edge_index, batch, x_b, edge_index_b, batch_b, lam,
           W1, b1, W2, b2, W3, b3, linW, linb):
    f32 = jnp.float32
    i32 = jnp.int32

    zpadn = jnp.zeros((N_PAD - N, D), f32)
    x_all = jnp.concatenate([x, zpadn, x_b, zpadn])

    def pad_edges(ei, src_off):
        src = jnp.concatenate(
            [ei[0] + i32(src_off), jnp.full((E_PAD - E,), src_off, i32)])
        dst = jnp.concatenate([ei[1], jnp.full((E_PAD - E,), N, i32)])
        return src.reshape(ER, CH), dst.reshape(ER, CH)

    srca, dsta = pad_edges(edge_index, 0)
    srcb, dstb = pad_edges(edge_index_b, N_PAD)
    src_all = jnp.concatenate([srca, srcb])
    dst_all = jnp.concatenate([dsta, dstb])

    bpadv = jnp.full((N_PAD - N,), NC * G, i32)
    bat2d = jnp.concatenate(
        [batch, bpadv, batch_b + i32(G), bpadv]).reshape(NC * N_PAD, 1)

    zlin = jnp.zeros((N_PAD,), f32)
    z32 = jnp.zeros((CH, QW), jnp.bfloat16)

    deg16 = _sc_deg(dst_all, zlin).reshape(NC * NS, N_PAD)

    *u1, deg2d = _tc_prep(x_all, deg16, W1)
    acc1 = _sc_edge(*u1, src_all, dst_all, z32)

    b1r, b2r, b3r = b1.reshape(1, H), b2.reshape(1, H), b3.reshape(1, H)
    u2 = _tc_mid(acc1, u1, deg2d, b1r, W2)
    acc2 = _sc_edge(*u2, src_all, dst_all, z32)

    u3 = _tc_mid(acc2, u2, deg2d, b2r, W3)
    acc3 = _sc_edge(*u3, src_all, dst_all, z32)

    pool_all, cnt_all = _tc_lastpool(acc3, u3, deg2d, b3r, bat2d)

    wpad = jnp.pad(linW, ((0, 0), (0, H - C)))
    bp = jnp.pad(linb, (0, H - C)).reshape(1, H)
    lam2 = jnp.reshape(lam, (1, 1)).astype(f32)

    out = _tc_final(pool_all[:G], cnt_all[:G],
                    pool_all[G:], cnt_all[G:], lam2, wpad, bp)
    return out[:, :C]
